# Initial kernel scaffold; baseline (speedup 1.0000x reference)
import jax
import jax.numpy as jnp
from jax.experimental import pallas as pl


def kernel(query, key, W, b):
    raise NotImplementedError("write your pallas kernel here")


# speedup vs baseline: 2.5179x; 2.5179x over previous
import jax
import jax.numpy as jnp
from jax.experimental import pallas as pl
from jax.experimental.pallas import tpu as pltpu

_QK_DIM = 512
_P = 256
_TOPK = 8
_SCALE = _QK_DIM ** (-0.5)
_NEG = float(jnp.finfo(jnp.float32).min)


def _fused_kernel(q_ref, k_ref, wt_ref, b_ref, w_out_ref, i_out_ref):
    qb = q_ref[0]
    kb = k_ref[0]
    wt = wt_ref[...]
    brow = b_ref[...]

    qh = jnp.dot(qb, wt, preferred_element_type=jnp.float32) + brow
    kh = jnp.dot(kb, wt, preferred_element_type=jnp.float32) + brow
    logits = jax.lax.dot_general(
        qh * _SCALE, kh, (((1,), (1,)), ((), ())),
        preferred_element_type=jnp.float32)

    row = jax.lax.broadcasted_iota(jnp.int32, (_P, _P), 0)
    col = jax.lax.broadcasted_iota(jnp.int32, (_P, _P), 1)
    logits = jnp.where(row == col, 1.0, logits)

    x = logits
    vals = []
    idxs = []
    for _ in range(_TOPK):
        mx = jnp.max(x, axis=-1, keepdims=True)
        hit = x == mx
        ix = jnp.min(jnp.where(hit, col, _P), axis=-1, keepdims=True)
        vals.append(mx)
        idxs.append(ix)
        x = jnp.where(col == ix, _NEG, x)

    v8 = jnp.concatenate(vals, axis=1)
    i8 = jnp.concatenate(idxs, axis=1)

    e = jnp.exp(v8 - v8[:, :1])
    w_out_ref[0] = e / jnp.sum(e, axis=1, keepdims=True)
    i_out_ref[0] = i8


@jax.jit
def kernel(query, key, W, b):
    batch = query.shape[0]
    wt = W.T
    brow = b.reshape(1, _QK_DIM)

    out = pl.pallas_call(
        _fused_kernel,
        grid=(batch,),
        in_specs=[
            pl.BlockSpec((1, _P, _QK_DIM), lambda i: (i, 0, 0)),
            pl.BlockSpec((1, _P, _QK_DIM), lambda i: (i, 0, 0)),
            pl.BlockSpec((_QK_DIM, _QK_DIM), lambda i: (0, 0)),
            pl.BlockSpec((1, _QK_DIM), lambda i: (0, 0)),
        ],
        out_specs=[
            pl.BlockSpec((1, _P, _TOPK), lambda i: (i, 0, 0)),
            pl.BlockSpec((1, _P, _TOPK), lambda i: (i, 0, 0)),
        ],
        out_shape=[
            jax.ShapeDtypeStruct((batch, _P, _TOPK), jnp.float32),
            jax.ShapeDtypeStruct((batch, _P, _TOPK), jnp.int32),
        ],
        compiler_params=pltpu.CompilerParams(
            dimension_semantics=("parallel",),
        ),
    )(query, key, wt, brow)
    return out[0], out[1]


# speedup vs baseline: 3.0101x; 1.1955x over previous
import jax
import jax.numpy as jnp
from jax.experimental import pallas as pl
from jax.experimental.pallas import tpu as pltpu

_QK_DIM = 512
_P = 256
_TOPK = 8
_SCALE = _QK_DIM ** (-0.5)
_NEG = float(jnp.finfo(jnp.float32).min)


def _fused_kernel(q_ref, k_ref, wt_ref, b_ref, w_out_ref, i_out_ref):
    qb = q_ref[0]
    kb = k_ref[0]
    wt = wt_ref[...]
    brow = b_ref[...]

    qh = jnp.dot(qb, wt, preferred_element_type=jnp.float32) + brow
    kh = jnp.dot(kb, wt, preferred_element_type=jnp.float32) + brow
    logits = jax.lax.dot_general(
        qh * _SCALE, kh, (((1,), (1,)), ((), ())),
        preferred_element_type=jnp.float32)

    row = jax.lax.broadcasted_iota(jnp.int32, (_P, _P), 0)
    col = jax.lax.broadcasted_iota(jnp.int32, (_P, _P), 1)
    logits = jnp.where(row == col, 1.0, logits)

    colf = col.astype(jnp.float32)
    x = logits
    vals = []
    idxs = []
    for _ in range(_TOPK):
        mx = jnp.max(x, axis=-1, keepdims=True)
        hit = x == mx
        ix = jnp.min(jnp.where(hit, colf, 256.0), axis=-1, keepdims=True)
        vals.append(mx)
        idxs.append(ix)
        x = jnp.where(colf == ix, _NEG, x)

    v8 = jnp.concatenate(vals, axis=1)
    i8 = jnp.concatenate(idxs, axis=1).astype(jnp.int32)

    e = jnp.exp(v8 - v8[:, :1])
    w_out_ref[0] = e / jnp.sum(e, axis=1, keepdims=True)
    i_out_ref[0] = i8


@jax.jit
def kernel(query, key, W, b):
    batch = query.shape[0]
    wt = W.T
    brow = b.reshape(1, _QK_DIM)

    out = pl.pallas_call(
        _fused_kernel,
        grid=(batch,),
        in_specs=[
            pl.BlockSpec((1, _P, _QK_DIM), lambda i: (i, 0, 0)),
            pl.BlockSpec((1, _P, _QK_DIM), lambda i: (i, 0, 0)),
            pl.BlockSpec((_QK_DIM, _QK_DIM), lambda i: (0, 0)),
            pl.BlockSpec((1, _QK_DIM), lambda i: (0, 0)),
        ],
        out_specs=[
            pl.BlockSpec((1, _P, _TOPK), lambda i: (i, 0, 0)),
            pl.BlockSpec((1, _P, _TOPK), lambda i: (i, 0, 0)),
        ],
        out_shape=[
            jax.ShapeDtypeStruct((batch, _P, _TOPK), jnp.float32),
            jax.ShapeDtypeStruct((batch, _P, _TOPK), jnp.int32),
        ],
        compiler_params=pltpu.CompilerParams(
            dimension_semantics=("parallel",),
        ),
    )(query, key, wt, brow)
    return out[0], out[1]


# speedup vs baseline: 4.1749x; 1.3869x over previous
import jax
import jax.numpy as jnp
from jax.experimental import pallas as pl
from jax.experimental.pallas import tpu as pltpu

_QK_DIM = 512
_P = 256
_TOPK = 8
_SCALE = _QK_DIM ** (-0.5)
_NEG = float(jnp.finfo(jnp.float32).min)


_BB = 2


def _fused_kernel(q_ref, k_ref, wt_ref, b_ref, w_out_ref, i_out_ref):
    wt = wt_ref[...]
    brow = b_ref[...]

    row = jax.lax.broadcasted_iota(jnp.int32, (_P, _P), 0)
    col = jax.lax.broadcasted_iota(jnp.int32, (_P, _P), 1)
    colf = col.astype(jnp.float32)

    for bi in range(_BB):
        qb = q_ref[bi]
        kb = k_ref[bi]

        qh = jnp.dot(qb, wt, preferred_element_type=jnp.float32) + brow
        kh = jnp.dot(kb, wt, preferred_element_type=jnp.float32) + brow
        logits = jax.lax.dot_general(
            qh * _SCALE, kh, (((1,), (1,)), ((), ())),
            preferred_element_type=jnp.float32)

        logits = jnp.where(row == col, 1.0, logits)

        x = logits
        vals = []
        idxs = []
        for _ in range(_TOPK):
            mx = jnp.max(x, axis=-1, keepdims=True)
            hit = x == mx
            ix = jnp.min(jnp.where(hit, colf, 256.0), axis=-1,
                         keepdims=True)
            vals.append(mx)
            idxs.append(ix)
            x = jnp.where(colf == ix, _NEG, x)

        v8 = jnp.concatenate(vals, axis=1)
        i8 = jnp.concatenate(idxs, axis=1).astype(jnp.int32)

        e = jnp.exp(v8 - v8[:, :1])
        w_out_ref[bi] = e / jnp.sum(e, axis=1, keepdims=True)
        i_out_ref[bi] = i8


@jax.jit
def kernel(query, key, W, b):
    batch = query.shape[0]
    wt = W.T
    brow = b.reshape(1, _QK_DIM)

    out = pl.pallas_call(
        _fused_kernel,
        grid=(batch // _BB,),
        in_specs=[
            pl.BlockSpec((_BB, _P, _QK_DIM), lambda i: (i, 0, 0)),
            pl.BlockSpec((_BB, _P, _QK_DIM), lambda i: (i, 0, 0)),
            pl.BlockSpec((_QK_DIM, _QK_DIM), lambda i: (0, 0)),
            pl.BlockSpec((1, _QK_DIM), lambda i: (0, 0)),
        ],
        out_specs=[
            pl.BlockSpec((_BB, _P, _TOPK), lambda i: (i, 0, 0)),
            pl.BlockSpec((_BB, _P, _TOPK), lambda i: (i, 0, 0)),
        ],
        out_shape=[
            jax.ShapeDtypeStruct((batch, _P, _TOPK), jnp.float32),
            jax.ShapeDtypeStruct((batch, _P, _TOPK), jnp.int32),
        ],
        compiler_params=pltpu.CompilerParams(
            dimension_semantics=("parallel",),
        ),
    )(query, key, wt, brow)
    return out[0], out[1]


# speedup vs baseline: 4.9479x; 1.1852x over previous
import jax
import jax.numpy as jnp
from jax.experimental import pallas as pl
from jax.experimental.pallas import tpu as pltpu

_QK_DIM = 512
_P = 256
_TOPK = 8
_SCALE = _QK_DIM ** (-0.5)
_NEG = float(jnp.finfo(jnp.float32).min)


_BB = 4


def _fused_kernel(q_ref, k_ref, wt_ref, b_ref, w_out_ref, i_out_ref):
    wt = wt_ref[...]
    brow = b_ref[...]

    row = jax.lax.broadcasted_iota(jnp.int32, (_P, _P), 0)
    col = jax.lax.broadcasted_iota(jnp.int32, (_P, _P), 1)
    colf = col.astype(jnp.float32)

    for bi in range(_BB):
        qb = q_ref[bi]
        kb = k_ref[bi]

        qh = jnp.dot(qb, wt, preferred_element_type=jnp.float32) + brow
        kh = jnp.dot(kb, wt, preferred_element_type=jnp.float32) + brow
        logits = jax.lax.dot_general(
            qh * _SCALE, kh, (((1,), (1,)), ((), ())),
            preferred_element_type=jnp.float32)

        logits = jnp.where(row == col, 1.0, logits)

        x = logits
        vals = []
        idxs = []
        for _ in range(_TOPK):
            mx = jnp.max(x, axis=-1, keepdims=True)
            hit = x == mx
            ix = jnp.min(jnp.where(hit, colf, 256.0), axis=-1,
                         keepdims=True)
            vals.append(mx)
            idxs.append(ix)
            x = jnp.where(colf == ix, _NEG, x)

        v8 = jnp.concatenate(vals, axis=1)
        i8 = jnp.concatenate(idxs, axis=1).astype(jnp.int32)

        e = jnp.exp(v8 - v8[:, :1])
        w_out_ref[bi] = e / jnp.sum(e, axis=1, keepdims=True)
        i_out_ref[bi] = i8


@jax.jit
def kernel(query, key, W, b):
    batch = query.shape[0]
    wt = W.T
    brow = b.reshape(1, _QK_DIM)

    out = pl.pallas_call(
        _fused_kernel,
        grid=(batch // _BB,),
        in_specs=[
            pl.BlockSpec((_BB, _P, _QK_DIM), lambda i: (i, 0, 0)),
            pl.BlockSpec((_BB, _P, _QK_DIM), lambda i: (i, 0, 0)),
            pl.BlockSpec((_QK_DIM, _QK_DIM), lambda i: (0, 0)),
            pl.BlockSpec((1, _QK_DIM), lambda i: (0, 0)),
        ],
        out_specs=[
            pl.BlockSpec((_BB, _P, _TOPK), lambda i: (i, 0, 0)),
            pl.BlockSpec((_BB, _P, _TOPK), lambda i: (i, 0, 0)),
        ],
        out_shape=[
            jax.ShapeDtypeStruct((batch, _P, _TOPK), jnp.float32),
            jax.ShapeDtypeStruct((batch, _P, _TOPK), jnp.int32),
        ],
        compiler_params=pltpu.CompilerParams(
            dimension_semantics=("parallel",),
        ),
    )(query, key, wt, brow)
    return out[0], out[1]


# speedup vs baseline: 5.1885x; 1.0486x over previous
import jax
import jax.numpy as jnp
from jax.experimental import pallas as pl
from jax.experimental.pallas import tpu as pltpu

_QK_DIM = 512
_P = 256
_TOPK = 8
_SCALE = _QK_DIM ** (-0.5)
_NEG = float(jnp.finfo(jnp.float32).min)


_BB = 8


def _fused_kernel(q_ref, k_ref, wt_ref, b_ref, w_out_ref, i_out_ref):
    wt = wt_ref[...]
    brow = b_ref[...]

    row = jax.lax.broadcasted_iota(jnp.int32, (_P, _P), 0)
    col = jax.lax.broadcasted_iota(jnp.int32, (_P, _P), 1)
    colf = col.astype(jnp.float32)

    for bi in range(_BB):
        qb = q_ref[bi]
        kb = k_ref[bi]

        qh = jnp.dot(qb, wt, preferred_element_type=jnp.float32) + brow
        kh = jnp.dot(kb, wt, preferred_element_type=jnp.float32) + brow
        logits = jax.lax.dot_general(
            qh * _SCALE, kh, (((1,), (1,)), ((), ())),
            preferred_element_type=jnp.float32)

        logits = jnp.where(row == col, 1.0, logits)

        x = logits
        vals = []
        idxs = []
        for _ in range(_TOPK):
            mx = jnp.max(x, axis=-1, keepdims=True)
            hit = x == mx
            ix = jnp.min(jnp.where(hit, colf, 256.0), axis=-1,
                         keepdims=True)
            vals.append(mx)
            idxs.append(ix)
            x = jnp.where(colf == ix, _NEG, x)

        v8 = jnp.concatenate(vals, axis=1)
        i8 = jnp.concatenate(idxs, axis=1).astype(jnp.int32)

        e = jnp.exp(v8 - v8[:, :1])
        w_out_ref[bi] = e / jnp.sum(e, axis=1, keepdims=True)
        i_out_ref[bi] = i8


@jax.jit
def kernel(query, key, W, b):
    batch = query.shape[0]
    wt = W.T
    brow = b.reshape(1, _QK_DIM)

    out = pl.pallas_call(
        _fused_kernel,
        grid=(batch // _BB,),
        in_specs=[
            pl.BlockSpec((_BB, _P, _QK_DIM), lambda i: (i, 0, 0)),
            pl.BlockSpec((_BB, _P, _QK_DIM), lambda i: (i, 0, 0)),
            pl.BlockSpec((_QK_DIM, _QK_DIM), lambda i: (0, 0)),
            pl.BlockSpec((1, _QK_DIM), lambda i: (0, 0)),
        ],
        out_specs=[
            pl.BlockSpec((_BB, _P, _TOPK), lambda i: (i, 0, 0)),
            pl.BlockSpec((_BB, _P, _TOPK), lambda i: (i, 0, 0)),
        ],
        out_shape=[
            jax.ShapeDtypeStruct((batch, _P, _TOPK), jnp.float32),
            jax.ShapeDtypeStruct((batch, _P, _TOPK), jnp.int32),
        ],
        compiler_params=pltpu.CompilerParams(
            dimension_semantics=("parallel",),
        ),
    )(query, key, wt, brow)
    return out[0], out[1]
